# baseline (speedup 1.0000x reference)
import jax
import jax.numpy as jnp
from jax import lax
from jax.experimental import pallas as pl
from jax.experimental.pallas import tpu as pltpu
from jax.experimental.pallas import tpu_sc as plsc
import functools

B = 4096
L = 50
LP = 56
D = 32
NC, NS = 2, 16
NW = NC * NS
BPW = B // NW


def _sc_pool(xp, w_word, w_bi, w_tri):
    mesh = plsc.VectorSubcoreMesh(core_axis_name="c", subcore_axis_name="s")

    @functools.partial(
        pl.kernel,
        out_type=jax.ShapeDtypeStruct((NW, BPW, 3 * D), jnp.float32),
        mesh=mesh,
        scratch_types=[
            pltpu.VMEM((BPW, LP), jnp.int32),
            pltpu.VMEM((LP, D), jnp.float32),
            pltpu.VMEM((LP, D), jnp.float32),
            pltpu.VMEM((LP, D), jnp.float32),
            pltpu.VMEM((LP, D), jnp.float32),
            pltpu.VMEM((LP, D), jnp.float32),
            pltpu.VMEM((LP, D), jnp.float32),
            pltpu.VMEM((BPW, 3 * D), jnp.float32),
            pltpu.SemaphoreType.DMA,
            pltpu.SemaphoreType.DMA,
        ],
        compiler_params=pltpu.CompilerParams(use_tc_tiling_on_sc=False),
    )
    def k(x_hbm, ww_hbm, wb_hbm, wt_hbm, out_hbm,
          idx_v, ga0, ga1, ga2, gb0, gb1, gb2, out_v, sa, sb):
        wid = lax.axis_index("s") * NC + lax.axis_index("c")
        pltpu.sync_copy(x_hbm.at[wid], idx_v)
        slots = ((ga0, ga1, ga2, sa), (gb0, gb1, gb2, sb))
        tables = (ww_hbm, wb_hbm, wt_hbm)

        def issue(b, slot):
            g0, g1, g2, sem = slots[slot]
            for t, g in zip(tables, (g0, g1, g2)):
                pltpu.make_async_copy(t.at[idx_v.at[b]], g, sem).start()

        def wait(slot):
            g0, g1, g2, sem = slots[slot]
            for g in (g0, g1, g2):
                pltpu.make_async_copy(tables[0].at[idx_v.at[0]], g,
                                      sem).wait()

        def reduce(b, slot):
            g0, g1, g2, _ = slots[slot]
            accs = [g[0, pl.ds(c, 16)] for g in (g0, g1, g2)
                    for c in (0, 16)]
            for l in range(1, L):
                for j, (g, c) in enumerate(
                        ((g, c) for g in (g0, g1, g2) for c in (0, 16))):
                    accs[j] = accs[j] + g[l, pl.ds(c, 16)]
            for j in range(6):
                out_v[b, pl.ds(16 * j, 16)] = accs[j]

        issue(0, 0)

        def b_body(i, carry):
            b = 2 * i
            issue(b + 1, 1)
            wait(0)
            reduce(b, 0)

            @pl.when(b + 2 < BPW)
            def _():
                issue(b + 2, 0)

            wait(1)
            reduce(b + 1, 1)
            return carry

        lax.fori_loop(0, BPW // 2, b_body, 0)
        pltpu.sync_copy(out_v, out_hbm.at[wid])

    return k(xp, w_word, w_bi, w_tri)


def _mlp_body(s_ref, w1_ref, b1_ref, w2_ref, b2_ref, o_ref):
    wf = jnp.dot(w1_ref[...], w2_ref[...],
                 preferred_element_type=jnp.float32) * (1.0 / L)
    bias = jnp.dot(b1_ref[...], w2_ref[...],
                   preferred_element_type=jnp.float32) + b2_ref[...]
    y = jnp.dot(s_ref[...], wf, preferred_element_type=jnp.float32) + bias
    o_ref[...] = jnp.maximum(y, 0.0)


def _mlp_tc(sums, w1, b1, w2, b2):
    return pl.pallas_call(
        _mlp_body,
        out_shape=jax.ShapeDtypeStruct((B, 32), jnp.float32),
    )(sums, w1, b1.reshape(1, -1), w2, b2.reshape(1, -1))


@jax.jit
def kernel(x, W_word, W_bi, W_tri, W1, b1, W2, b2):
    xp = jnp.pad(x, ((0, 0), (0, LP - L)))
    xp = xp.reshape(NW, BPW, LP)
    sums = _sc_pool(xp, W_word, W_bi, W_tri)
    sums = sums.reshape(B, 3 * D)
    return _mlp_tc(sums, W1, b1, W2, b2)
